# baseline (device time: 30293 ns/iter reference)
import jax
import jax.numpy as jnp
from jax import lax
from jax.experimental import pallas as pl
from jax.experimental.pallas import tpu as pltpu

N_DEV = 4


def kernel(x, w_mat):
    m_per, k = x.shape
    _, n_per = w_mat.shape

    def body(x_ref, w_ref, out_ref, comm_ref, send_sems, recv_sems):
        my_pos = lax.axis_index("i")
        left = lax.rem(my_pos + N_DEV - 1, N_DEV)
        right = lax.rem(my_pos + 1, N_DEV)

        barrier_sem = pltpu.get_barrier_semaphore()
        for nbr in [left, right]:
            pl.semaphore_signal(
                barrier_sem, inc=1,
                device_id=(nbr,), device_id_type=pl.DeviceIdType.MESH,
            )
        pl.semaphore_wait(barrier_sem, 2)

        w = w_ref[...].astype(jnp.bfloat16)
        comm_ref[0] = x_ref[...].astype(jnp.bfloat16)

        out_ref[pl.ds(my_pos * m_per, m_per), :] = jnp.dot(
            comm_ref[0], w, preferred_element_type=jnp.float32
        )

        for h in range(N_DEV - 1):
            send_slot = h % 2
            recv_slot = (h + 1) % 2
            rdma = pltpu.make_async_remote_copy(
                src_ref=comm_ref.at[send_slot],
                dst_ref=comm_ref.at[recv_slot],
                send_sem=send_sems.at[send_slot],
                recv_sem=recv_sems.at[recv_slot],
                device_id=(right,),
                device_id_type=pl.DeviceIdType.MESH,
            )
            rdma.start()
            rdma.wait()

            origin = lax.rem(my_pos + N_DEV - 1 - h, N_DEV)
            out_ref[pl.ds(origin * m_per, m_per), :] = jnp.dot(
                comm_ref[recv_slot], w, preferred_element_type=jnp.float32
            )

    out_shape = jax.ShapeDtypeStruct((N_DEV * m_per, n_per), jnp.float32)
    return pl.pallas_call(
        body,
        out_shape=out_shape,
        in_specs=[
            pl.BlockSpec(memory_space=pltpu.VMEM),
            pl.BlockSpec(memory_space=pltpu.VMEM),
        ],
        out_specs=pl.BlockSpec(memory_space=pltpu.VMEM),
        scratch_shapes=[
            pltpu.VMEM((2, m_per, k), jnp.bfloat16),
            pltpu.SemaphoreType.DMA((2,)),
            pltpu.SemaphoreType.DMA((2,)),
        ],
        compiler_params=pltpu.CompilerParams(collective_id=0),
    )(x, w_mat)


# device time: 20632 ns/iter; 1.4683x vs baseline; 1.4683x over previous
import jax
import jax.numpy as jnp
from jax import lax
from jax.experimental import pallas as pl
from jax.experimental.pallas import tpu as pltpu

N_DEV = 4


def kernel(x, w_mat):
    m_per, k = x.shape
    _, n_per = w_mat.shape
    m_half = m_per // 2

    def body(x_ref, w_ref, out_ref,
             comm_r, comm_l, send_r, recv_r, send_l, recv_l):
        my_pos = lax.axis_index("i")
        left = lax.rem(my_pos + N_DEV - 1, N_DEV)
        right = lax.rem(my_pos + 1, N_DEV)

        barrier_sem = pltpu.get_barrier_semaphore()
        for nbr in [left, right]:
            pl.semaphore_signal(
                barrier_sem, inc=1,
                device_id=(nbr,), device_id_type=pl.DeviceIdType.MESH,
            )
        pl.semaphore_wait(barrier_sem, 2)

        w = w_ref[...].astype(jnp.bfloat16)
        comm_r[0] = x_ref[:m_half, :].astype(jnp.bfloat16)
        comm_l[0] = x_ref[m_half:, :].astype(jnp.bfloat16)

        def hop_gemm(slot, h):
            org_r = lax.rem(my_pos + N_DEV - h, N_DEV)
            org_l = lax.rem(my_pos + h, N_DEV)
            out_ref[pl.ds(org_r * m_per, m_half), :] = jnp.dot(
                comm_r[slot], w, preferred_element_type=jnp.float32
            )
            out_ref[pl.ds(org_l * m_per + m_half, m_half), :] = jnp.dot(
                comm_l[slot], w, preferred_element_type=jnp.float32
            )

        for h in range(N_DEV - 1):
            s = h % 2
            r = (h + 1) % 2
            rdma_r = pltpu.make_async_remote_copy(
                src_ref=comm_r.at[s], dst_ref=comm_r.at[r],
                send_sem=send_r.at[s], recv_sem=recv_r.at[r],
                device_id=(right,), device_id_type=pl.DeviceIdType.MESH,
            )
            rdma_l = pltpu.make_async_remote_copy(
                src_ref=comm_l.at[s], dst_ref=comm_l.at[r],
                send_sem=send_l.at[s], recv_sem=recv_l.at[r],
                device_id=(left,), device_id_type=pl.DeviceIdType.MESH,
            )
            rdma_r.start()
            rdma_l.start()
            hop_gemm(s, h)
            rdma_r.wait()
            rdma_l.wait()

        hop_gemm((N_DEV - 1) % 2, N_DEV - 1)

    out_shape = jax.ShapeDtypeStruct((N_DEV * m_per, n_per), jnp.float32)
    return pl.pallas_call(
        body,
        out_shape=out_shape,
        in_specs=[
            pl.BlockSpec(memory_space=pltpu.VMEM),
            pl.BlockSpec(memory_space=pltpu.VMEM),
        ],
        out_specs=pl.BlockSpec(memory_space=pltpu.VMEM),
        scratch_shapes=[
            pltpu.VMEM((2, m_half, k), jnp.bfloat16),
            pltpu.VMEM((2, m_half, k), jnp.bfloat16),
            pltpu.SemaphoreType.DMA((2,)),
            pltpu.SemaphoreType.DMA((2,)),
            pltpu.SemaphoreType.DMA((2,)),
            pltpu.SemaphoreType.DMA((2,)),
        ],
        compiler_params=pltpu.CompilerParams(collective_id=0),
    )(x, w_mat)


# device time: 17515 ns/iter; 1.7295x vs baseline; 1.1780x over previous
import jax
import jax.numpy as jnp
from jax import lax
from jax.experimental import pallas as pl
from jax.experimental.pallas import tpu as pltpu

N_DEV = 4


def kernel(x, w_mat):
    m_per, k = x.shape
    _, n_per = w_mat.shape
    m_half = m_per // 2

    def body(x_ref, w_ref, out_ref,
             mine, from_l, from_r, diag_a, diag_b, send_sems, recv_sems):
        my_pos = lax.axis_index("i")
        left = lax.rem(my_pos + N_DEV - 1, N_DEV)
        right = lax.rem(my_pos + 1, N_DEV)
        diag = lax.rem(my_pos + 2, N_DEV)

        barrier_sem = pltpu.get_barrier_semaphore()
        for nbr in [left, right]:
            pl.semaphore_signal(
                barrier_sem, inc=1,
                device_id=(nbr,), device_id_type=pl.DeviceIdType.MESH,
            )
        pl.semaphore_wait(barrier_sem, 2)

        mine[0] = x_ref[:m_half, :].astype(jnp.bfloat16)
        mine[1] = x_ref[m_half:, :].astype(jnp.bfloat16)

        def rcopy(src, dst, sem_idx, dev):
            return pltpu.make_async_remote_copy(
                src_ref=src, dst_ref=dst,
                send_sem=send_sems.at[sem_idx], recv_sem=recv_sems.at[sem_idx],
                device_id=(dev,), device_id_type=pl.DeviceIdType.MESH,
            )

        sends = [
            rcopy(mine.at[0], from_l.at[0], 0, right),
            rcopy(mine.at[1], from_r.at[1], 1, left),
            rcopy(mine.at[1], from_l.at[1], 2, right),
            rcopy(mine.at[0], from_r.at[0], 3, left),
        ]
        for s in sends:
            s.start()

        w = w_ref[...].astype(jnp.bfloat16)
        out_ref[pl.ds(my_pos * m_per, m_half), :] = jnp.dot(
            mine[0], w, preferred_element_type=jnp.float32)
        out_ref[pl.ds(my_pos * m_per + m_half, m_half), :] = jnp.dot(
            mine[1], w, preferred_element_type=jnp.float32)

        rcopy(from_l.at[0], from_l.at[0], 0, left).wait_recv()
        fwd_r = rcopy(from_l.at[0], diag_a, 4, right)
        fwd_r.start()
        rcopy(from_r.at[1], from_r.at[1], 1, right).wait_recv()
        fwd_l = rcopy(from_r.at[1], diag_b, 5, left)
        fwd_l.start()

        rcopy(from_l.at[1], from_l.at[1], 2, left).wait_recv()
        out_ref[pl.ds(left * m_per, m_half), :] = jnp.dot(
            from_l[0], w, preferred_element_type=jnp.float32)
        out_ref[pl.ds(left * m_per + m_half, m_half), :] = jnp.dot(
            from_l[1], w, preferred_element_type=jnp.float32)

        rcopy(from_r.at[0], from_r.at[0], 3, right).wait_recv()
        out_ref[pl.ds(right * m_per, m_half), :] = jnp.dot(
            from_r[0], w, preferred_element_type=jnp.float32)
        out_ref[pl.ds(right * m_per + m_half, m_half), :] = jnp.dot(
            from_r[1], w, preferred_element_type=jnp.float32)

        rcopy(diag_a, diag_a, 4, left).wait_recv()
        out_ref[pl.ds(diag * m_per, m_half), :] = jnp.dot(
            diag_a[...], w, preferred_element_type=jnp.float32)
        rcopy(diag_b, diag_b, 5, right).wait_recv()
        out_ref[pl.ds(diag * m_per + m_half, m_half), :] = jnp.dot(
            diag_b[...], w, preferred_element_type=jnp.float32)

        for s in sends:
            s.wait_send()
        fwd_r.wait_send()
        fwd_l.wait_send()

    out_shape = jax.ShapeDtypeStruct((N_DEV * m_per, n_per), jnp.float32)
    return pl.pallas_call(
        body,
        out_shape=out_shape,
        in_specs=[
            pl.BlockSpec(memory_space=pltpu.VMEM),
            pl.BlockSpec(memory_space=pltpu.VMEM),
        ],
        out_specs=pl.BlockSpec(memory_space=pltpu.VMEM),
        scratch_shapes=[
            pltpu.VMEM((2, m_half, k), jnp.bfloat16),
            pltpu.VMEM((2, m_half, k), jnp.bfloat16),
            pltpu.VMEM((2, m_half, k), jnp.bfloat16),
            pltpu.VMEM((m_half, k), jnp.bfloat16),
            pltpu.VMEM((m_half, k), jnp.bfloat16),
            pltpu.SemaphoreType.DMA((6,)),
            pltpu.SemaphoreType.DMA((6,)),
        ],
        compiler_params=pltpu.CompilerParams(collective_id=0),
    )(x, w_mat)
